# Initial kernel scaffold; baseline (speedup 1.0000x reference)
import jax
import jax.numpy as jnp
from jax.experimental import pallas as pl


def kernel(boxes, scores):
    raise NotImplementedError("write your pallas kernel here")


# speedup vs baseline: 14.0667x; 14.0667x over previous
import functools

import jax
import jax.numpy as jnp
from jax import lax
from jax.experimental import pallas as pl
from jax.experimental.pallas import tpu as pltpu
from jax.experimental.pallas import tpu_sc as plsc

N = 20000
SCORE_THRESH = 0.5
IOU_THRESH = 0.8
MAX_DET = 100

NSUB = 16
LANES = 16
PER_SUB = 1280
CHUNKS = PER_SUB // LANES
NPAD = NSUB * PER_SUB

NEG = -1e30
BIG = 1e30


def _nms_kernel(x1_h, y1_h, x2_h, y2_h, sc_h, out_h,
                x1_v, y1_v, x2_v, y2_v, sv, cand_v, allc_v, out_v, shared):
    s = lax.axis_index("s")
    base = s * PER_SUB
    base_f = base.astype(jnp.float32)
    io = lax.iota(jnp.int32, LANES)
    iof = io.astype(jnp.float32)
    zeros_i = jnp.zeros((LANES,), jnp.int32)
    neg16 = jnp.full((LANES,), NEG, jnp.float32)
    big16 = jnp.full((LANES,), BIG, jnp.float32)

    pltpu.sync_copy(x1_h.at[pl.ds(base, PER_SUB)], x1_v)
    pltpu.sync_copy(y1_h.at[pl.ds(base, PER_SUB)], y1_v)
    pltpu.sync_copy(x2_h.at[pl.ds(base, PER_SUB)], x2_v)
    pltpu.sync_copy(y2_h.at[pl.ds(base, PER_SUB)], y2_v)
    pltpu.sync_copy(sc_h.at[pl.ds(base, PER_SUB)], sv)

    def init_chunk(k, carry):
        rmax, ridx = carry
        sl = pl.ds(k * LANES, LANES)
        v = sv[sl]
        v = jnp.where(v > SCORE_THRESH, v, NEG)
        sv[sl] = v
        gidx = base_f + (k * LANES).astype(jnp.float32) + iof
        ridx = jnp.where(v > rmax, gidx, ridx)
        rmax = jnp.maximum(rmax, v)
        return rmax, ridx

    carry0 = lax.fori_loop(0, CHUNKS, init_chunk, (neg16, big16))

    def pick(t, carry):
        rmax, ridx = carry
        smax = jnp.max(rmax)
        lidx = jnp.min(jnp.where(rmax == smax, ridx, big16))
        li = jnp.clip(lidx - base_f, 0.0, float(PER_SUB - 1)).astype(jnp.int32)
        liv = zeros_i + li
        gx1 = plsc.load_gather(x1_v, [liv])
        gy1 = plsc.load_gather(y1_v, [liv])
        gx2 = plsc.load_gather(x2_v, [liv])
        gy2 = plsc.load_gather(y2_v, [liv])
        row = (jnp.where(io == 0, smax, 0.0)
               + jnp.where(io == 1, lidx, 0.0)
               + jnp.where(io == 2, gx1, 0.0)
               + jnp.where(io == 3, gy1, 0.0)
               + jnp.where(io == 4, gx2, 0.0)
               + jnp.where(io == 5, gy2, 0.0))
        cand_v[...] = row
        pltpu.sync_copy(cand_v, shared.at[pl.ds(s * LANES, LANES)])
        plsc.subcore_barrier()
        pltpu.sync_copy(shared, allc_v)
        plsc.subcore_barrier()
        stride = io * LANES
        scv = plsc.load_gather(allc_v, [stride])
        idv = plsc.load_gather(allc_v, [stride + 1])
        gmax = jnp.max(scv)
        ok = gmax > NEG
        widx = jnp.min(jnp.where(scv == gmax, idv, big16))
        cw = jnp.min(jnp.where((scv == gmax) & (idv == widx), iof, big16)
                     ).astype(jnp.int32)
        cwv = zeros_i + cw * LANES
        bx1 = plsc.load_gather(allc_v, [cwv + 2])
        by1 = plsc.load_gather(allc_v, [cwv + 3])
        bx2 = plsc.load_gather(allc_v, [cwv + 4])
        by2 = plsc.load_gather(allc_v, [cwv + 5])

        @pl.when(s == 0)
        def _():
            orow = (jnp.where(io == 0, bx1, 0.0)
                    + jnp.where(io == 1, by1, 0.0)
                    + jnp.where(io == 2, bx2, 0.0)
                    + jnp.where(io == 3, by2, 0.0)
                    + jnp.where(io == 4, gmax, 0.0))
            out_v[pl.ds(t * LANES, LANES)] = orow * jnp.where(ok, 1.0, 0.0)

        area_a = (bx2 - bx1) * (by2 - by1)

        def supp_chunk(k, carry):
            nrun, nidx = carry
            sl = pl.ds(k * LANES, LANES)
            sv_k = sv[sl]
            x1k = x1_v[sl]
            y1k = y1_v[sl]
            x2k = x2_v[sl]
            y2k = y2_v[sl]
            xx1 = jnp.maximum(bx1, x1k)
            yy1 = jnp.maximum(by1, y1k)
            xx2 = jnp.minimum(bx2, x2k)
            yy2 = jnp.minimum(by2, y2k)
            inter = jnp.maximum(xx2 - xx1, 0.0) * jnp.maximum(yy2 - yy1, 0.0)
            area_b = (x2k - x1k) * (y2k - y1k)
            union = area_a + area_b - inter
            iou = inter / jnp.maximum(union, 1e-9)
            supp = (iou > IOU_THRESH) & ok
            s2 = jnp.where(supp, NEG, sv_k)
            sv[sl] = s2
            gidx = base_f + (k * LANES).astype(jnp.float32) + iof
            nidx = jnp.where(s2 > nrun, gidx, nidx)
            nrun = jnp.maximum(nrun, s2)
            return nrun, nidx

        return lax.fori_loop(0, CHUNKS, supp_chunk, (neg16, big16))

    lax.fori_loop(0, MAX_DET, pick, carry0)

    @pl.when(s == 0)
    def _():
        pltpu.sync_copy(out_v, out_h)


@jax.jit
def kernel(boxes, scores):
    pad = NPAD - N
    x1 = jnp.pad(boxes[:, 0], (0, pad))
    y1 = jnp.pad(boxes[:, 1], (0, pad))
    x2 = jnp.pad(boxes[:, 2], (0, pad))
    y2 = jnp.pad(boxes[:, 3], (0, pad))
    sc = jnp.pad(scores, (0, pad), constant_values=-1.0)

    nms = functools.partial(
        pl.kernel,
        out_type=jax.ShapeDtypeStruct((MAX_DET * LANES,), jnp.float32),
        mesh=plsc.VectorSubcoreMesh(
            core_axis_name="c", subcore_axis_name="s", num_cores=1),
        compiler_params=pltpu.CompilerParams(needs_layout_passes=False),
        scratch_types=[
            pltpu.VMEM((PER_SUB,), jnp.float32),
            pltpu.VMEM((PER_SUB,), jnp.float32),
            pltpu.VMEM((PER_SUB,), jnp.float32),
            pltpu.VMEM((PER_SUB,), jnp.float32),
            pltpu.VMEM((PER_SUB,), jnp.float32),
            pltpu.VMEM((LANES,), jnp.float32),
            pltpu.VMEM((NSUB * LANES,), jnp.float32),
            pltpu.VMEM((MAX_DET * LANES,), jnp.float32),
            pltpu.VMEM_SHARED((NSUB * LANES,), jnp.float32),
        ],
    )(_nms_kernel)
    out = nms(x1, y1, x2, y2, sc)
    return out.reshape(MAX_DET, LANES)[:, :5]
